# baseline (speedup 1.0000x reference)
import functools

import jax
import jax.numpy as jnp
from jax.experimental import pallas as pl
from jax.experimental.pallas import tpu as pltpu

F = 256
H = 8
D = 32
HD = H * D

TM = 512
TE = 640


def _split(x):
    hi = x.astype(jnp.bfloat16)
    lo = (x - hi.astype(jnp.float32)).astype(jnp.bfloat16)
    return hi, lo


def _dense_kernel(x_ref, wv_ref, bv_ref, ws_ref, bs_ref, wd_ref, bd_ref,
                  al_ref, ar_ref,
                  vh_ref, vl_ref, elh_ref, ell_ref, erh_ref, erl_ref):
    x = x_ref[...]
    v = jnp.dot(x, wv_ref[...], preferred_element_type=jnp.float32) + bv_ref[...]
    vh_ref[...], vl_ref[...] = _split(v)
    rows = jax.lax.broadcasted_iota(jnp.int32, (HD, H), 0) // D
    cols = jax.lax.broadcasted_iota(jnp.int32, (HD, H), 1)
    hmask = (rows == cols).astype(jnp.float32)
    a_l = al_ref[...] * hmask
    a_r = ar_ref[...] * hmask
    p_l = jnp.dot(ws_ref[...], a_l, preferred_element_type=jnp.float32)
    p_r = jnp.dot(wd_ref[...], a_r, preferred_element_type=jnp.float32)
    el = (jnp.dot(x, p_l, preferred_element_type=jnp.float32)
          + jnp.dot(bs_ref[...], a_l, preferred_element_type=jnp.float32))
    er = (jnp.dot(x, p_r, preferred_element_type=jnp.float32)
          + jnp.dot(bd_ref[...], a_r, preferred_element_type=jnp.float32))
    elh_ref[...], ell_ref[...] = _split(el)
    erh_ref[...], erl_ref[...] = _split(er)


def _edge_kernel(src_c_ref, dst_c_ref, dst_r_ref,
                 vh_ref, vl_ref, elh_ref, ell_ref, erh_ref, erl_ref,
                 num_ref, den_ref, *, n_pad, n_tiles):
    j = pl.program_id(0)

    @pl.when(j == 0)
    def _init():
        num_ref[...] = jnp.zeros_like(num_ref)
        den_ref[...] = jnp.zeros_like(den_ref)

    src_c = src_c_ref[0]
    dst_c = dst_c_ref[0]
    dst_r = dst_r_ref[0]

    nchunks = n_pad // TM

    def dot32(a, b):
        return jnp.dot(a, b, preferred_element_type=jnp.float32)

    def gather_body(k, carry):
        vsrc, elg, erg = carry
        base = k * TM
        ids = base + jax.lax.broadcasted_iota(jnp.int32, (TE, TM), 1)
        m_s = (src_c == ids).astype(jnp.bfloat16)
        m_d = (dst_c == ids).astype(jnp.bfloat16)
        sl = pl.ds(base, TM)
        vsrc = vsrc + dot32(m_s, vh_ref[sl, :]) + dot32(m_s, vl_ref[sl, :])
        elg = elg + dot32(m_s, elh_ref[sl, :]) + dot32(m_s, ell_ref[sl, :])
        erg = erg + dot32(m_d, erh_ref[sl, :]) + dot32(m_d, erl_ref[sl, :])
        return vsrc, elg, erg

    vsrc, elg, erg = jax.lax.fori_loop(
        0, nchunks, gather_body,
        (jnp.zeros((TE, HD), jnp.float32),
         jnp.zeros((TE, H), jnp.float32),
         jnp.zeros((TE, H), jnp.float32)))

    e = elg + erg
    e = jnp.where(e > 0, e, 0.2 * e)
    c = (jnp.max(elh_ref[...].astype(jnp.float32))
         + jnp.max(erh_ref[...].astype(jnp.float32)))
    ee = jnp.exp(e - c)

    hrow = jax.lax.broadcasted_iota(jnp.int32, (H, HD), 0)
    hcol = jax.lax.broadcasted_iota(jnp.int32, (H, HD), 1) // D
    expand = (hrow == hcol).astype(jnp.float32)
    msg = vsrc * dot32(ee, expand)
    msg_hi, msg_lo = _split(msg)
    ee_hi, ee_lo = _split(ee)

    def scatter_body(k, _):
        base = k * TM
        ids = base + jax.lax.broadcasted_iota(jnp.int32, (TM, TE), 0)
        m_t = (ids == dst_r).astype(jnp.bfloat16)
        sl = pl.ds(base, TM)
        num_ref[sl, :] += dot32(m_t, msg_hi) + dot32(m_t, msg_lo)
        den_ref[sl, :] += dot32(m_t, ee_hi) + dot32(m_t, ee_lo)
        return 0

    jax.lax.fori_loop(0, nchunks, scatter_body, 0)

    @pl.when(j == n_tiles - 1)
    def _normalize():
        den = jnp.maximum(den_ref[...], 1e-30)
        denx = jnp.dot(den, expand, preferred_element_type=jnp.float32)
        num_ref[...] = num_ref[...] / denx


def kernel(feat, edge_index, W_src, b_src, W_dst, b_dst, W_v, b_v, attn_l, attn_r):
    n = feat.shape[0]
    e_total = edge_index.shape[1]
    n_pad = ((n + TM - 1) // TM) * TM
    n_tiles = e_total // TE
    nb = n_pad // 256

    feat_p = jnp.pad(feat, ((0, n_pad - n), (0, 0)))
    al_col = attn_l.reshape(HD, 1)
    ar_col = attn_r.reshape(HD, 1)
    bs = b_src.reshape(1, HD)
    bd = b_dst.reshape(1, HD)
    bv = b_v.reshape(1, HD)

    bspec = lambda blk: pl.BlockSpec(blk, lambda i: (i, 0))
    cspec = lambda blk: pl.BlockSpec(blk, lambda i: (0, 0))

    vh, vl, elh, ell, erh, erl = pl.pallas_call(
        _dense_kernel,
        grid=(nb,),
        in_specs=[
            bspec((256, F)),
            cspec((F, HD)), cspec((1, HD)),
            cspec((F, HD)), cspec((1, HD)),
            cspec((F, HD)), cspec((1, HD)),
            cspec((HD, 1)), cspec((HD, 1)),
        ],
        out_specs=[
            bspec((256, HD)), bspec((256, HD)),
            bspec((256, H)), bspec((256, H)),
            bspec((256, H)), bspec((256, H)),
        ],
        out_shape=[
            jax.ShapeDtypeStruct((n_pad, HD), jnp.bfloat16),
            jax.ShapeDtypeStruct((n_pad, HD), jnp.bfloat16),
            jax.ShapeDtypeStruct((n_pad, H), jnp.bfloat16),
            jax.ShapeDtypeStruct((n_pad, H), jnp.bfloat16),
            jax.ShapeDtypeStruct((n_pad, H), jnp.bfloat16),
            jax.ShapeDtypeStruct((n_pad, H), jnp.bfloat16),
        ],
        compiler_params=pltpu.CompilerParams(
            dimension_semantics=("parallel",)),
    )(feat_p, W_v, bv, W_src, bs, W_dst, bd, al_col, ar_col)

    src = edge_index[0]
    dst = edge_index[1]
    src_c = src.reshape(n_tiles, TE, 1)
    dst_c = dst.reshape(n_tiles, TE, 1)
    dst_r = dst.reshape(n_tiles, 1, TE)

    num, _den = pl.pallas_call(
        functools.partial(_edge_kernel, n_pad=n_pad, n_tiles=n_tiles),
        grid=(n_tiles,),
        in_specs=[
            pl.BlockSpec((1, TE, 1), lambda j: (j, 0, 0)),
            pl.BlockSpec((1, TE, 1), lambda j: (j, 0, 0)),
            pl.BlockSpec((1, 1, TE), lambda j: (j, 0, 0)),
            cspec((n_pad, HD)), cspec((n_pad, HD)),
            cspec((n_pad, H)), cspec((n_pad, H)),
            cspec((n_pad, H)), cspec((n_pad, H)),
        ],
        out_specs=[
            cspec((n_pad, HD)),
            cspec((n_pad, H)),
        ],
        out_shape=[
            jax.ShapeDtypeStruct((n_pad, HD), jnp.float32),
            jax.ShapeDtypeStruct((n_pad, H), jnp.float32),
        ],
        compiler_params=pltpu.CompilerParams(
            dimension_semantics=("arbitrary",)),
    )(src_c, dst_c, dst_r, vh, vl, elh, ell, erh, erl)

    return num[:n]


# speedup vs baseline: 1.9926x; 1.9926x over previous
import functools

import jax
import jax.numpy as jnp
from jax.experimental import pallas as pl
from jax.experimental.pallas import tpu as pltpu

F = 256
H = 8
D = 32
HD = H * D

TM = 1024
TE = 640


def _dense_kernel(x_ref, wv_ref, bv_ref, ws_ref, bs_ref, wd_ref, bd_ref,
                  al_ref, ar_ref, v_ref, el_ref, er_ref):
    x = x_ref[...]
    v_ref[...] = jnp.dot(x, wv_ref[...], preferred_element_type=jnp.float32) + bv_ref[...]
    rows = jax.lax.broadcasted_iota(jnp.int32, (HD, H), 0) // D
    cols = jax.lax.broadcasted_iota(jnp.int32, (HD, H), 1)
    hmask = (rows == cols).astype(jnp.float32)
    a_l = al_ref[...] * hmask
    a_r = ar_ref[...] * hmask
    p_l = jnp.dot(ws_ref[...], a_l, preferred_element_type=jnp.float32)
    p_r = jnp.dot(wd_ref[...], a_r, preferred_element_type=jnp.float32)
    el_ref[...] = (jnp.dot(x, p_l, preferred_element_type=jnp.float32)
                   + jnp.dot(bs_ref[...], a_l, preferred_element_type=jnp.float32))
    er_ref[...] = (jnp.dot(x, p_r, preferred_element_type=jnp.float32)
                   + jnp.dot(bd_ref[...], a_r, preferred_element_type=jnp.float32))


def _edge_kernel(src_c_ref, dst_c_ref, dst_r_ref, v_ref, el_ref, er_ref,
                 num_ref, den_ref, c_ref, *, n_pad, n_tiles):
    j = pl.program_id(0)

    @pl.when(j == 0)
    def _init():
        num_ref[...] = jnp.zeros_like(num_ref)
        den_ref[...] = jnp.zeros_like(den_ref)
        c_ref[0, 0] = jnp.max(el_ref[...]) + jnp.max(er_ref[...])

    src_c = src_c_ref[0]
    dst_c = dst_c_ref[0]
    dst_r = dst_r_ref[0]

    nchunks = n_pad // TM

    def gather_body(k, carry):
        vsrc, elg, erg = carry
        base = k * TM
        ids = base + jax.lax.broadcasted_iota(jnp.int32, (TE, TM), 1)
        m_s = (src_c == ids).astype(jnp.float32)
        m_d = (dst_c == ids).astype(jnp.float32)
        vc = v_ref[pl.ds(base, TM), :]
        vsrc = vsrc + jnp.dot(m_s, vc, preferred_element_type=jnp.float32)
        elg = elg + jnp.dot(m_s, el_ref[pl.ds(base, TM), :],
                            preferred_element_type=jnp.float32)
        erg = erg + jnp.dot(m_d, er_ref[pl.ds(base, TM), :],
                            preferred_element_type=jnp.float32)
        return vsrc, elg, erg

    vsrc, elg, erg = jax.lax.fori_loop(
        0, nchunks, gather_body,
        (jnp.zeros((TE, HD), jnp.float32),
         jnp.zeros((TE, H), jnp.float32),
         jnp.zeros((TE, H), jnp.float32)))

    e = elg + erg
    e = jnp.where(e > 0, e, 0.2 * e)
    ee = jnp.exp(e - c_ref[0, 0])

    hrow = jax.lax.broadcasted_iota(jnp.int32, (H, HD), 0)
    hcol = jax.lax.broadcasted_iota(jnp.int32, (H, HD), 1) // D
    expand = (hrow == hcol).astype(jnp.float32)
    msg = vsrc * jnp.dot(ee, expand, preferred_element_type=jnp.float32)

    def scatter_body(k, _):
        base = k * TM
        ids = base + jax.lax.broadcasted_iota(jnp.int32, (TM, TE), 0)
        m_t = (ids == dst_r).astype(jnp.float32)
        num_ref[pl.ds(base, TM), :] += jnp.dot(m_t, msg,
                                               preferred_element_type=jnp.float32)
        den_ref[pl.ds(base, TM), :] += jnp.dot(m_t, ee,
                                               preferred_element_type=jnp.float32)
        return 0

    jax.lax.fori_loop(0, nchunks, scatter_body, 0)

    @pl.when(j == n_tiles - 1)
    def _normalize():
        den = jnp.maximum(den_ref[...], 1e-30)
        denx = jnp.dot(den, expand, preferred_element_type=jnp.float32)
        num_ref[...] = num_ref[...] / denx


def kernel(feat, edge_index, W_src, b_src, W_dst, b_dst, W_v, b_v, attn_l, attn_r):
    n = feat.shape[0]
    e_total = edge_index.shape[1]
    n_pad = ((n + TM - 1) // TM) * TM
    n_tiles = e_total // TE
    nb = n_pad // 256

    feat_p = jnp.pad(feat, ((0, n_pad - n), (0, 0)))
    al_col = attn_l.reshape(HD, 1)
    ar_col = attn_r.reshape(HD, 1)
    bs = b_src.reshape(1, HD)
    bd = b_dst.reshape(1, HD)
    bv = b_v.reshape(1, HD)

    v, el, er = pl.pallas_call(
        _dense_kernel,
        grid=(nb,),
        in_specs=[
            pl.BlockSpec((256, F), lambda i: (i, 0)),
            pl.BlockSpec((F, HD), lambda i: (0, 0)),
            pl.BlockSpec((1, HD), lambda i: (0, 0)),
            pl.BlockSpec((F, HD), lambda i: (0, 0)),
            pl.BlockSpec((1, HD), lambda i: (0, 0)),
            pl.BlockSpec((F, HD), lambda i: (0, 0)),
            pl.BlockSpec((1, HD), lambda i: (0, 0)),
            pl.BlockSpec((HD, 1), lambda i: (0, 0)),
            pl.BlockSpec((HD, 1), lambda i: (0, 0)),
        ],
        out_specs=[
            pl.BlockSpec((256, HD), lambda i: (i, 0)),
            pl.BlockSpec((256, H), lambda i: (i, 0)),
            pl.BlockSpec((256, H), lambda i: (i, 0)),
        ],
        out_shape=[
            jax.ShapeDtypeStruct((n_pad, HD), jnp.float32),
            jax.ShapeDtypeStruct((n_pad, H), jnp.float32),
            jax.ShapeDtypeStruct((n_pad, H), jnp.float32),
        ],
        compiler_params=pltpu.CompilerParams(
            dimension_semantics=("parallel",)),
    )(feat_p, W_v, bv, W_src, bs, W_dst, bd, al_col, ar_col)

    src = edge_index[0]
    dst = edge_index[1]
    src_c = src.reshape(n_tiles, TE, 1)
    dst_c = dst.reshape(n_tiles, TE, 1)
    dst_r = dst.reshape(n_tiles, 1, TE)

    num, _den = pl.pallas_call(
        functools.partial(_edge_kernel, n_pad=n_pad, n_tiles=n_tiles),
        grid=(n_tiles,),
        in_specs=[
            pl.BlockSpec((1, TE, 1), lambda j: (j, 0, 0)),
            pl.BlockSpec((1, TE, 1), lambda j: (j, 0, 0)),
            pl.BlockSpec((1, 1, TE), lambda j: (j, 0, 0)),
            pl.BlockSpec((n_pad, HD), lambda j: (0, 0)),
            pl.BlockSpec((n_pad, H), lambda j: (0, 0)),
            pl.BlockSpec((n_pad, H), lambda j: (0, 0)),
        ],
        out_specs=[
            pl.BlockSpec((n_pad, HD), lambda j: (0, 0)),
            pl.BlockSpec((n_pad, H), lambda j: (0, 0)),
        ],
        out_shape=[
            jax.ShapeDtypeStruct((n_pad, HD), jnp.float32),
            jax.ShapeDtypeStruct((n_pad, H), jnp.float32),
        ],
        compiler_params=pltpu.CompilerParams(
            dimension_semantics=("arbitrary",)),
        scratch_shapes=[pltpu.SMEM((1, 1), jnp.float32)],
    )(src_c, dst_c, dst_r, v, el, er)

    return num[:n]


# speedup vs baseline: 2.2180x; 1.1131x over previous
import functools

import jax
import jax.numpy as jnp
from jax.experimental import pallas as pl
from jax.experimental.pallas import tpu as pltpu

F = 256
H = 8
D = 32
HD = H * D

TM = 1024
TE = 1280


def _dense_kernel(x_ref, wv_ref, bv_ref, ws_ref, bs_ref, wd_ref, bd_ref,
                  al_ref, ar_ref, v_ref, el_ref, er_ref):
    x = x_ref[...]
    v_ref[...] = jnp.dot(x, wv_ref[...], preferred_element_type=jnp.float32) + bv_ref[...]
    rows = jax.lax.broadcasted_iota(jnp.int32, (HD, H), 0) // D
    cols = jax.lax.broadcasted_iota(jnp.int32, (HD, H), 1)
    hmask = (rows == cols).astype(jnp.float32)
    a_l = al_ref[...] * hmask
    a_r = ar_ref[...] * hmask
    p_l = jnp.dot(ws_ref[...], a_l, preferred_element_type=jnp.float32)
    p_r = jnp.dot(wd_ref[...], a_r, preferred_element_type=jnp.float32)
    el_ref[...] = (jnp.dot(x, p_l, preferred_element_type=jnp.float32)
                   + jnp.dot(bs_ref[...], a_l, preferred_element_type=jnp.float32))
    er_ref[...] = (jnp.dot(x, p_r, preferred_element_type=jnp.float32)
                   + jnp.dot(bd_ref[...], a_r, preferred_element_type=jnp.float32))


def _edge_kernel(src_c_ref, dst_c_ref, dst_r_ref, v_ref, el_ref, er_ref,
                 num_ref, den_ref, c_ref, *, n_pad, n_tiles):
    j = pl.program_id(0)

    @pl.when(j == 0)
    def _init():
        num_ref[...] = jnp.zeros_like(num_ref)
        den_ref[...] = jnp.zeros_like(den_ref)
        c_ref[0, 0] = jnp.max(el_ref[...]) + jnp.max(er_ref[...])

    src_c = src_c_ref[0]
    dst_c = dst_c_ref[0]
    dst_r = dst_r_ref[0]

    nchunks = n_pad // TM

    def gather_body(k, carry):
        vsrc, elg, erg = carry
        base = k * TM
        ids = base + jax.lax.broadcasted_iota(jnp.int32, (TE, TM), 1)
        m_s = (src_c == ids).astype(jnp.float32)
        m_d = (dst_c == ids).astype(jnp.float32)
        vc = v_ref[pl.ds(base, TM), :]
        vsrc = vsrc + jnp.dot(m_s, vc, preferred_element_type=jnp.float32)
        elg = elg + jnp.dot(m_s, el_ref[pl.ds(base, TM), :],
                            preferred_element_type=jnp.float32)
        erg = erg + jnp.dot(m_d, er_ref[pl.ds(base, TM), :],
                            preferred_element_type=jnp.float32)
        return vsrc, elg, erg

    vsrc, elg, erg = jax.lax.fori_loop(
        0, nchunks, gather_body,
        (jnp.zeros((TE, HD), jnp.float32),
         jnp.zeros((TE, H), jnp.float32),
         jnp.zeros((TE, H), jnp.float32)))

    e = elg + erg
    e = jnp.where(e > 0, e, 0.2 * e)
    ee = jnp.exp(e - c_ref[0, 0])

    hrow = jax.lax.broadcasted_iota(jnp.int32, (H, HD), 0)
    hcol = jax.lax.broadcasted_iota(jnp.int32, (H, HD), 1) // D
    expand = (hrow == hcol).astype(jnp.float32)
    msg = vsrc * jnp.dot(ee, expand, preferred_element_type=jnp.float32)

    def scatter_body(k, _):
        base = k * TM
        ids = base + jax.lax.broadcasted_iota(jnp.int32, (TM, TE), 0)
        m_t = (ids == dst_r).astype(jnp.float32)
        num_ref[pl.ds(base, TM), :] += jnp.dot(m_t, msg,
                                               preferred_element_type=jnp.float32)
        den_ref[pl.ds(base, TM), :] += jnp.dot(m_t, ee,
                                               preferred_element_type=jnp.float32)
        return 0

    jax.lax.fori_loop(0, nchunks, scatter_body, 0)

    @pl.when(j == n_tiles - 1)
    def _normalize():
        den = jnp.maximum(den_ref[...], 1e-30)
        denx = jnp.dot(den, expand, preferred_element_type=jnp.float32)
        num_ref[...] = num_ref[...] / denx


def kernel(feat, edge_index, W_src, b_src, W_dst, b_dst, W_v, b_v, attn_l, attn_r):
    n = feat.shape[0]
    e_total = edge_index.shape[1]
    n_pad = ((n + TM - 1) // TM) * TM
    n_tiles = e_total // TE
    nb = n_pad // 256

    feat_p = jnp.pad(feat, ((0, n_pad - n), (0, 0)))
    al_col = attn_l.reshape(HD, 1)
    ar_col = attn_r.reshape(HD, 1)
    bs = b_src.reshape(1, HD)
    bd = b_dst.reshape(1, HD)
    bv = b_v.reshape(1, HD)

    v, el, er = pl.pallas_call(
        _dense_kernel,
        grid=(nb,),
        in_specs=[
            pl.BlockSpec((256, F), lambda i: (i, 0)),
            pl.BlockSpec((F, HD), lambda i: (0, 0)),
            pl.BlockSpec((1, HD), lambda i: (0, 0)),
            pl.BlockSpec((F, HD), lambda i: (0, 0)),
            pl.BlockSpec((1, HD), lambda i: (0, 0)),
            pl.BlockSpec((F, HD), lambda i: (0, 0)),
            pl.BlockSpec((1, HD), lambda i: (0, 0)),
            pl.BlockSpec((HD, 1), lambda i: (0, 0)),
            pl.BlockSpec((HD, 1), lambda i: (0, 0)),
        ],
        out_specs=[
            pl.BlockSpec((256, HD), lambda i: (i, 0)),
            pl.BlockSpec((256, H), lambda i: (i, 0)),
            pl.BlockSpec((256, H), lambda i: (i, 0)),
        ],
        out_shape=[
            jax.ShapeDtypeStruct((n_pad, HD), jnp.float32),
            jax.ShapeDtypeStruct((n_pad, H), jnp.float32),
            jax.ShapeDtypeStruct((n_pad, H), jnp.float32),
        ],
        compiler_params=pltpu.CompilerParams(
            dimension_semantics=("parallel",)),
    )(feat_p, W_v, bv, W_src, bs, W_dst, bd, al_col, ar_col)

    src = edge_index[0]
    dst = edge_index[1]
    src_c = src.reshape(n_tiles, TE, 1)
    dst_c = dst.reshape(n_tiles, TE, 1)
    dst_r = dst.reshape(n_tiles, 1, TE)

    num, _den = pl.pallas_call(
        functools.partial(_edge_kernel, n_pad=n_pad, n_tiles=n_tiles),
        grid=(n_tiles,),
        in_specs=[
            pl.BlockSpec((1, TE, 1), lambda j: (j, 0, 0)),
            pl.BlockSpec((1, TE, 1), lambda j: (j, 0, 0)),
            pl.BlockSpec((1, 1, TE), lambda j: (j, 0, 0)),
            pl.BlockSpec((n_pad, HD), lambda j: (0, 0)),
            pl.BlockSpec((n_pad, H), lambda j: (0, 0)),
            pl.BlockSpec((n_pad, H), lambda j: (0, 0)),
        ],
        out_specs=[
            pl.BlockSpec((n_pad, HD), lambda j: (0, 0)),
            pl.BlockSpec((n_pad, H), lambda j: (0, 0)),
        ],
        out_shape=[
            jax.ShapeDtypeStruct((n_pad, HD), jnp.float32),
            jax.ShapeDtypeStruct((n_pad, H), jnp.float32),
        ],
        compiler_params=pltpu.CompilerParams(
            dimension_semantics=("arbitrary",)),
        scratch_shapes=[pltpu.SMEM((1, 1), jnp.float32)],
    )(src_c, dst_c, dst_r, v, el, er)

    return num[:n]
